# initial kernel scaffold (unmeasured)
import jax
import jax.numpy as jnp
from jax import lax
from jax.experimental import pallas as pl
from jax.experimental.pallas import tpu as pltpu


def kernel(x, assign, W1, W2):
    t, d = x.shape
    e_per, _, f = W1.shape
    assign2d = assign.reshape(t, 1)

    def body(x_ref, a_ref, w1_ref, w2_ref, out_ref,
             xr_ref, ar_ref, psend_ref, precv_ref,
             send_sems, recv_sems):
        my_x = lax.axis_index("x")
        my_y = lax.axis_index("y")
        my_z = lax.axis_index("z")
        nbr = (my_x, my_y, 1 - my_z)

        barrier_sem = pltpu.get_barrier_semaphore()
        pl.semaphore_signal(barrier_sem, inc=1, device_id=nbr,
                            device_id_type=pl.DeviceIdType.MESH)
        pl.semaphore_wait(barrier_sem, 1)

        rdma_x = pltpu.make_async_remote_copy(
            src_ref=x_ref, dst_ref=xr_ref,
            send_sem=send_sems.at[0], recv_sem=recv_sems.at[0],
            device_id=nbr, device_id_type=pl.DeviceIdType.MESH)
        rdma_x.start()
        rdma_a = pltpu.make_async_remote_copy(
            src_ref=a_ref, dst_ref=ar_ref,
            send_sem=send_sems.at[1], recv_sem=recv_sems.at[1],
            device_id=nbr, device_id_type=pl.DeviceIdType.MESH)
        rdma_a.start()

        def experts_masked(xin, ain):
            acc = jnp.zeros((t, d), jnp.float32)
            for e in range(e_per):
                eg = my_z * e_per + e
                h = jnp.maximum(
                    jnp.dot(xin, w1_ref[e], preferred_element_type=jnp.float32),
                    0.0)
                y = jnp.dot(h, w2_ref[e], preferred_element_type=jnp.float32)
                acc = acc + jnp.where(ain == eg, y, 0.0)
            return acc

        out_ref[...] = experts_masked(x_ref[...], a_ref[...])

        rdma_x.wait()
        rdma_a.wait()

        psend_ref[...] = experts_masked(xr_ref[...], ar_ref[...])
        rdma_p = pltpu.make_async_remote_copy(
            src_ref=psend_ref, dst_ref=precv_ref,
            send_sem=send_sems.at[2], recv_sem=recv_sems.at[2],
            device_id=nbr, device_id_type=pl.DeviceIdType.MESH)
        rdma_p.start()
        rdma_p.wait()

        out_ref[...] = out_ref[...] + precv_ref[...]

    return pl.pallas_call(
        body,
        out_shape=jax.ShapeDtypeStruct((t, d), jnp.float32),
        in_specs=[pl.BlockSpec(memory_space=pltpu.VMEM)] * 4,
        out_specs=pl.BlockSpec(memory_space=pltpu.VMEM),
        scratch_shapes=[
            pltpu.VMEM((t, d), jnp.float32),
            pltpu.VMEM((t, 1), jnp.int32),
            pltpu.VMEM((t, d), jnp.float32),
            pltpu.VMEM((t, d), jnp.float32),
            pltpu.SemaphoreType.DMA((3,)),
            pltpu.SemaphoreType.DMA((3,)),
        ],
        compiler_params=pltpu.CompilerParams(collective_id=0),
    )(x, assign2d, W1, W2)


# baseline (device time: 141537 ns/iter reference)
import jax
import jax.numpy as jnp
from jax import lax
from jax.experimental import pallas as pl
from jax.experimental.pallas import tpu as pltpu


def kernel(x, assign, W1, W2):
    t, d = x.shape
    e_per, _, f = W1.shape
    assign2d = assign.reshape(t, 1)

    def body(x_ref, a_ref, w1_ref, w2_ref, out_ref,
             xr_ref, ar_ref, psend_ref, precv_ref,
             send_sems, recv_sems):
        my_x = lax.axis_index("x")
        my_y = lax.axis_index("y")
        my_z = lax.axis_index("z")
        nbr = (my_x, my_y, 1 - my_z)

        barrier_sem = pltpu.get_barrier_semaphore()
        pl.semaphore_signal(barrier_sem, inc=1, device_id=nbr,
                            device_id_type=pl.DeviceIdType.MESH)
        pl.semaphore_wait(barrier_sem, 1)

        rdma_x = pltpu.make_async_remote_copy(
            src_ref=x_ref, dst_ref=xr_ref,
            send_sem=send_sems.at[0], recv_sem=recv_sems.at[0],
            device_id=nbr, device_id_type=pl.DeviceIdType.MESH)
        rdma_x.start()
        rdma_a = pltpu.make_async_remote_copy(
            src_ref=a_ref, dst_ref=ar_ref,
            send_sem=send_sems.at[1], recv_sem=recv_sems.at[1],
            device_id=nbr, device_id_type=pl.DeviceIdType.MESH)
        rdma_a.start()

        TC = 512
        FC = 1024

        def experts_masked(x_in, a_in, dst):
            for tc in range(0, t, TC):
                rows = pl.ds(tc, TC)
                xin = x_in[rows, :]
                ain = a_in[rows, :]
                acc = jnp.zeros((TC, d), jnp.float32)
                for e in range(e_per):
                    eg = my_z * e_per + e
                    y = jnp.zeros((TC, d), jnp.float32)
                    for fc in range(0, f, FC):
                        h = jnp.maximum(
                            jnp.dot(xin, w1_ref[e, :, pl.ds(fc, FC)],
                                    preferred_element_type=jnp.float32),
                            0.0)
                        y = y + jnp.dot(h, w2_ref[e, pl.ds(fc, FC), :],
                                        preferred_element_type=jnp.float32)
                    acc = acc + jnp.where(ain == eg, y, 0.0)
                dst[rows, :] = acc

        experts_masked(x_ref, a_ref, out_ref)

        rdma_x.wait()
        rdma_a.wait()

        experts_masked(xr_ref, ar_ref, psend_ref)
        rdma_p = pltpu.make_async_remote_copy(
            src_ref=psend_ref, dst_ref=precv_ref,
            send_sem=send_sems.at[2], recv_sem=recv_sems.at[2],
            device_id=nbr, device_id_type=pl.DeviceIdType.MESH)
        rdma_p.start()
        rdma_p.wait()

        out_ref[...] = out_ref[...] + precv_ref[...]

    return pl.pallas_call(
        body,
        out_shape=jax.ShapeDtypeStruct((t, d), jnp.float32),
        in_specs=[pl.BlockSpec(memory_space=pltpu.VMEM)] * 4,
        out_specs=pl.BlockSpec(memory_space=pltpu.VMEM),
        scratch_shapes=[
            pltpu.VMEM((t, d), jnp.float32),
            pltpu.VMEM((t, 1), jnp.int32),
            pltpu.VMEM((t, d), jnp.float32),
            pltpu.VMEM((t, d), jnp.float32),
            pltpu.SemaphoreType.DMA((3,)),
            pltpu.SemaphoreType.DMA((3,)),
        ],
        compiler_params=pltpu.CompilerParams(
            collective_id=0, vmem_limit_bytes=100 * 1024 * 1024),
    )(x, assign2d, W1, W2)


# device time: 82918 ns/iter; 1.7070x vs baseline; 1.7070x over previous
import jax
import jax.numpy as jnp
from jax import lax
from jax.experimental import pallas as pl
from jax.experimental.pallas import tpu as pltpu

BF = jnp.bfloat16
TC_A = 512
TC_B = 256
FC = 1024


def kernel(x, assign, W1, W2):
    t, d = x.shape
    e_per, _, f = W1.shape
    assign2d = assign.reshape(t, 1)
    n_b = t // TC_B

    def body(x_ref, a_ref, w1_ref, w2_ref, out_ref,
             xs_ref, xr_ref, ar_ref, psend_ref, precv_ref,
             send_sems, recv_sems):
        my_x = lax.axis_index("x")
        my_y = lax.axis_index("y")
        my_z = lax.axis_index("z")
        nbr = (my_x, my_y, 1 - my_z)

        barrier_sem = pltpu.get_barrier_semaphore()
        pl.semaphore_signal(barrier_sem, inc=1, device_id=nbr,
                            device_id_type=pl.DeviceIdType.MESH)
        pl.semaphore_wait(barrier_sem, 1)

        xs_ref[...] = x_ref[...].astype(BF)
        rdma_x = pltpu.make_async_remote_copy(
            src_ref=xs_ref, dst_ref=xr_ref,
            send_sem=send_sems.at[0], recv_sem=recv_sems.at[0],
            device_id=nbr, device_id_type=pl.DeviceIdType.MESH)
        rdma_x.start()
        rdma_a = pltpu.make_async_remote_copy(
            src_ref=a_ref, dst_ref=ar_ref,
            send_sem=send_sems.at[1], recv_sem=recv_sems.at[1],
            device_id=nbr, device_id_type=pl.DeviceIdType.MESH)
        rdma_a.start()

        def chunk_masked(xin_bf, ain):
            tc = xin_bf.shape[0]
            acc = jnp.zeros((tc, d), jnp.float32)
            for e in range(e_per):
                eg = my_z * e_per + e
                y = jnp.zeros((tc, d), jnp.float32)
                for fc in range(0, f, FC):
                    h = jnp.maximum(
                        jnp.dot(xin_bf, w1_ref[e, :, pl.ds(fc, FC)].astype(BF),
                                preferred_element_type=jnp.float32),
                        0.0)
                    y = y + jnp.dot(h.astype(BF),
                                    w2_ref[e, pl.ds(fc, FC), :].astype(BF),
                                    preferred_element_type=jnp.float32)
                acc = acc + jnp.where(ain == eg, y, 0.0)
            return acc

        for tc in range(0, t, TC_A):
            rows = pl.ds(tc, TC_A)
            out_ref[rows, :] = chunk_masked(xs_ref[rows, :], a_ref[rows, :])

        rdma_x.wait()
        rdma_a.wait()

        rdma_p = []
        for ci in range(n_b):
            rows = pl.ds(ci * TC_B, TC_B)
            psend_ref[rows, :] = chunk_masked(
                xr_ref[rows, :], ar_ref[rows, :]).astype(BF)
            r = pltpu.make_async_remote_copy(
                src_ref=psend_ref.at[rows],
                dst_ref=precv_ref.at[rows],
                send_sem=send_sems.at[2 + ci], recv_sem=recv_sems.at[2 + ci],
                device_id=nbr, device_id_type=pl.DeviceIdType.MESH)
            r.start()
            rdma_p.append(r)

        for ci in range(n_b):
            rdma_p[ci].wait()
            rows = pl.ds(ci * TC_B, TC_B)
            out_ref[rows, :] = (out_ref[rows, :]
                                + precv_ref[rows, :].astype(jnp.float32))

    return pl.pallas_call(
        body,
        out_shape=jax.ShapeDtypeStruct((t, d), jnp.float32),
        in_specs=[pl.BlockSpec(memory_space=pltpu.VMEM)] * 4,
        out_specs=pl.BlockSpec(memory_space=pltpu.VMEM),
        scratch_shapes=[
            pltpu.VMEM((t, d), BF),
            pltpu.VMEM((t, d), BF),
            pltpu.VMEM((t, 1), jnp.int32),
            pltpu.VMEM((t, d), BF),
            pltpu.VMEM((t, d), BF),
            pltpu.SemaphoreType.DMA((2 + n_b,)),
            pltpu.SemaphoreType.DMA((2 + n_b,)),
        ],
        compiler_params=pltpu.CompilerParams(
            collective_id=0, vmem_limit_bytes=100 * 1024 * 1024),
    )(x, assign2d, W1, W2)
